# baseline (speedup 1.0000x reference)
import functools

import jax
import jax.numpy as jnp
from jax import lax
from jax.experimental import pallas as pl
from jax.experimental.pallas import tpu as pltpu
from jax.experimental.pallas import tpu_sc as plsc

N = 10000
E = 320000
DF = 128
DE = 16
H = 32
L = 3

NC = 2
NS = 16
NW = NC * NS
EPAD = 327680
EPT = EPAD // NW
CH = 128
NCH = EPT // CH
NP = 10240
RPT = NP // NS
PRB = 12800
PRB4 = PRB // 4


def _edge_pass(h, ea, srcm, dstm):
    mesh = plsc.VectorSubcoreMesh(core_axis_name="c", subcore_axis_name="s")

    @functools.partial(
        pl.kernel,
        out_type=jax.ShapeDtypeStruct((NC * NP, H), jnp.float32),
        mesh=mesh,
        scratch_types=[
            pltpu.VMEM((NCH, CH), jnp.int32),
            pltpu.VMEM((NCH, CH), jnp.int32),
            pltpu.VMEM((CH, H), jnp.float32),
            pltpu.VMEM((CH, H), jnp.float32),
            pltpu.VMEM((CH, H), jnp.float32),
            pltpu.VMEM((CH, H), jnp.float32),
            pltpu.VMEM((CH, H), jnp.float32),
            pltpu.VMEM((CH, H), jnp.float32),
            pltpu.VMEM((RPT, H), jnp.float32),
            pltpu.VMEM_SHARED((NP, H), jnp.float32),
            pltpu.SemaphoreType.DMA,
            pltpu.SemaphoreType.DMA,
            pltpu.SemaphoreType.DMA,
            pltpu.SemaphoreType.DMA,
            pltpu.SemaphoreType.DMA,
            pltpu.SemaphoreType.DMA,
        ],
        compiler_params=pltpu.CompilerParams(use_tc_tiling_on_sc=False),
    )
    def edge_kernel(h_hbm, ea_hbm, srcm_hbm, dstm_hbm,
                    out_hbm,
                    srcm_v, dstm_v,
                    hs0, hs1, ea0, ea1, ms0, ms1,
                    buf_v, agg_sh,
                    gsem0, gsem1, esem0, esem1, ssem0, ssem1):
        c = lax.axis_index("c")
        s = lax.axis_index("s")
        wid = c * NS + s
        base = wid * EPT

        zv = jnp.zeros((16,), jnp.float32)

        def zrow(r, carry):
            buf_v[r, pl.ds(0, 16)] = zv
            buf_v[r, pl.ds(16, 16)] = zv
            return carry

        lax.fori_loop(0, RPT, zrow, 0, unroll=8)
        pltpu.sync_copy(buf_v, agg_sh.at[pl.ds(s * RPT, RPT)])

        pltpu.sync_copy(srcm_hbm.at[wid], srcm_v)
        pltpu.sync_copy(dstm_hbm.at[wid], dstm_v)
        plsc.subcore_barrier()

        def ea_slice(e0):
            i = e0 // PRB
            rem = e0 - i * PRB
            cb = rem // PRB4
            row0 = i * PRB4 + (rem - cb * PRB4)
            return row0, cb * H

        def issue(k, eab, hsb, esem, gsem):
            row0, col0 = ea_slice(base + k * CH)
            pltpu.async_copy(
                ea_hbm.at[pl.ds(row0, CH), pl.ds(col0, H)], eab, esem)
            pltpu.async_copy(h_hbm.at[srcm_v.at[k]], hsb, gsem)

        def wait_loads(k, eab, hsb, esem, gsem):
            row0, col0 = ea_slice(base + k * CH)
            pltpu.make_async_copy(
                ea_hbm.at[pl.ds(row0, CH), pl.ds(col0, H)], eab, esem).wait()
            pltpu.make_async_copy(h_hbm.at[srcm_v.at[k]], hsb, gsem).wait()

        def compute(eab, hsb, msb):
            def row(r, rc):
                msb[r, pl.ds(0, 16)] = jnp.maximum(
                    hsb[r, pl.ds(0, 16)] + eab[r, pl.ds(0, 16)], 0.0)
                msb[r, pl.ds(16, 16)] = jnp.maximum(
                    hsb[r, pl.ds(16, 16)] + eab[r, pl.ds(16, 16)], 0.0)
                return rc

            lax.fori_loop(0, CH, row, 0, unroll=8)

        issue(0, ea0, hs0, esem0, gsem0)
        issue(1, ea1, hs1, esem1, gsem1)

        def pair(i, carry):
            k0 = 2 * i
            k1 = 2 * i + 1

            wait_loads(k0, ea0, hs0, esem0, gsem0)

            @pl.when(i > 0)
            def _():
                pltpu.make_async_copy(
                    ms0, agg_sh.at[dstm_v.at[k0 - 2]], ssem0).wait()

            compute(ea0, hs0, ms0)

            @pl.when(k0 + 2 < NCH)
            def _():
                issue(k0 + 2, ea0, hs0, esem0, gsem0)

            pltpu.async_copy(ms0, agg_sh.at[dstm_v.at[k0]], ssem0, add=True)

            wait_loads(k1, ea1, hs1, esem1, gsem1)

            @pl.when(i > 0)
            def _():
                pltpu.make_async_copy(
                    ms1, agg_sh.at[dstm_v.at[k1 - 2]], ssem1).wait()

            compute(ea1, hs1, ms1)

            @pl.when(k1 + 2 < NCH)
            def _():
                issue(k1 + 2, ea1, hs1, esem1, gsem1)

            pltpu.async_copy(ms1, agg_sh.at[dstm_v.at[k1]], ssem1, add=True)
            return carry

        lax.fori_loop(0, NCH // 2, pair, 0)

        pltpu.make_async_copy(ms0, agg_sh.at[dstm_v.at[NCH - 2]], ssem0).wait()
        pltpu.make_async_copy(ms1, agg_sh.at[dstm_v.at[NCH - 1]], ssem1).wait()

        plsc.subcore_barrier()
        pltpu.sync_copy(agg_sh.at[pl.ds(s * RPT, RPT)], buf_v)
        pltpu.sync_copy(buf_v, out_hbm.at[pl.ds(wid * RPT, RPT)])

    return edge_kernel(h, ea, srcm, dstm)


def _proj_node(x, W, b):
    def body(x_ref, w_ref, b_ref, o_ref):
        o_ref[...] = jnp.dot(x_ref[...], w_ref[...],
                             preferred_element_type=jnp.float32) + b_ref[...]

    return pl.pallas_call(
        body, out_shape=jax.ShapeDtypeStruct((N, H), jnp.float32),
    )(x, W, b.reshape(1, H))


def _proj_edge(eaT, W, b):
    RB = PRB
    G = E // RB

    RB4 = RB // 4

    def body(a_ref, w_ref, b_ref, o_ref):
        y = lax.dot_general(a_ref[...], w_ref[...],
                            (((0,), (0,)), ((), ())),
                            preferred_element_type=jnp.float32) + b_ref[...]
        o_ref[...] = jnp.concatenate(
            [y[0:RB4], y[RB4:2 * RB4], y[2 * RB4:3 * RB4], y[3 * RB4:]],
            axis=1)

    return pl.pallas_call(
        body,
        grid=(G,),
        in_specs=[
            pl.BlockSpec((DE, RB), lambda i: (0, i)),
            pl.BlockSpec((DE, H), lambda i: (0, 0)),
            pl.BlockSpec((1, H), lambda i: (0, 0)),
        ],
        out_specs=pl.BlockSpec((RB // 4, 4 * H), lambda i: (i, 0)),
        out_shape=jax.ShapeDtypeStruct((EPAD // 4, 4 * H), jnp.float32),
    )(eaT, W, b.reshape(1, H))


def _dense_layer(h, a0, a1, W1i, b1i, W2i, b2i, gi, bi):
    def body(h_ref, a0_ref, a1_ref, w1, b1, w2, b2, g, bt, o_ref):
        z = h_ref[...] + a0_ref[...] + a1_ref[...]
        t = jnp.maximum(
            jnp.dot(z, w1[...], preferred_element_type=jnp.float32) + b1[...], 0.0)
        z2 = jnp.dot(t, w2[...], preferred_element_type=jnp.float32) + b2[...]
        mu = jnp.mean(z2, axis=0, keepdims=True)
        var = jnp.mean((z2 - mu) ** 2, axis=0, keepdims=True)
        o_ref[...] = jnp.maximum(
            g[...] * (z2 - mu) * lax.rsqrt(var + 1e-5) + bt[...], 0.0)

    return pl.pallas_call(
        body, out_shape=jax.ShapeDtypeStruct((N, H), jnp.float32),
    )(h, a0, a1, W1i, b1i.reshape(1, 2 * H), W2i, b2i.reshape(1, H),
      gi.reshape(1, H), bi.reshape(1, H))


def _dense_final(h, a0, a1, W1i, b1i, W2i, b2i, gi, bi, Wl2, bl2, Wl3, bl3):
    def body(h_ref, a0_ref, a1_ref, w1, b1, w2, b2, g, bt,
             wl2, l2b, wl3, l3b, o_ref):
        z = h_ref[...] + a0_ref[...] + a1_ref[...]
        t = jnp.maximum(
            jnp.dot(z, w1[...], preferred_element_type=jnp.float32) + b1[...], 0.0)
        z2 = jnp.dot(t, w2[...], preferred_element_type=jnp.float32) + b2[...]
        mu = jnp.mean(z2, axis=0, keepdims=True)
        var = jnp.mean((z2 - mu) ** 2, axis=0, keepdims=True)
        hp = jnp.maximum(
            g[...] * (z2 - mu) * lax.rsqrt(var + 1e-5) + bt[...], 0.0)
        gx = jnp.sum(hp, axis=0, keepdims=True)
        p = jnp.maximum(
            jnp.dot(gx, wl2[...], preferred_element_type=jnp.float32) + l2b[...], 0.0)
        o_ref[...] = jnp.dot(p, wl3[...],
                             preferred_element_type=jnp.float32) + l3b[...]

    return pl.pallas_call(
        body, out_shape=jax.ShapeDtypeStruct((1, 1), jnp.float32),
    )(h, a0, a1, W1i, b1i.reshape(1, 2 * H), W2i, b2i.reshape(1, H),
      gi.reshape(1, H), bi.reshape(1, H),
      Wl2, bl2.reshape(1, H // 2), Wl3, bl3.reshape(1, 1))


def kernel(x, edge_attr, edge_index, W_node, b_node, W_edge, b_edge,
           W1, b1, W2, b2, gamma, beta, Wl2, bl2, Wl3, bl3):
    srcp = jnp.concatenate(
        [edge_index[0], jnp.zeros((EPAD - E,), jnp.int32)])
    dstp = jnp.concatenate(
        [edge_index[1], N + jnp.arange(EPAD - E, dtype=jnp.int32) % (NP - N)])
    srcm = srcp.reshape(NW, NCH, CH)
    dstm = dstp.reshape(NW, NCH, CH)
    h = _proj_node(x, W_node, b_node)
    ea = _proj_edge(edge_attr.T, W_edge, b_edge)
    for i in range(L):
        parts = _edge_pass(h, ea, srcm, dstm)
        a0 = parts[:N]
        a1 = parts[NP:NP + N]
        if i < L - 1:
            h = _dense_layer(h, a0, a1, W1[i], b1[i], W2[i], b2[i],
                             gamma[i], beta[i])
        else:
            res = _dense_final(h, a0, a1, W1[i], b1[i], W2[i], b2[i],
                               gamma[i], beta[i], Wl2, bl2, Wl3, bl3)
    return res.reshape(1)


# speedup vs baseline: 1.0123x; 1.0123x over previous
import functools

import jax
import jax.numpy as jnp
from jax import lax
from jax.experimental import pallas as pl
from jax.experimental.pallas import tpu as pltpu
from jax.experimental.pallas import tpu_sc as plsc

N = 10000
E = 320000
DF = 128
DE = 16
H = 32
L = 3

NC = 2
NS = 16
NW = NC * NS
EPAD = 327680
EPT = EPAD // NW
CH = 128
NCH = EPT // CH
NP = 10240
RPT = NP // NS
PRB = 12800
PRB4 = PRB // 4


def _edge_pass(h, ea, srcm, dstm):
    mesh = plsc.VectorSubcoreMesh(core_axis_name="c", subcore_axis_name="s")

    @functools.partial(
        pl.kernel,
        out_type=jax.ShapeDtypeStruct((NC * NP, H), jnp.float32),
        mesh=mesh,
        scratch_types=[
            pltpu.VMEM((NCH, CH), jnp.int32),
            pltpu.VMEM((NCH, CH), jnp.int32),
            pltpu.VMEM((CH, H), jnp.float32),
            pltpu.VMEM((CH, H), jnp.float32),
            pltpu.VMEM((CH, H), jnp.float32),
            pltpu.VMEM((CH, H), jnp.float32),
            pltpu.VMEM((CH, H), jnp.float32),
            pltpu.VMEM((CH, H), jnp.float32),
            pltpu.VMEM((RPT, H), jnp.float32),
            pltpu.VMEM_SHARED((NP, H), jnp.float32),
            pltpu.SemaphoreType.DMA,
            pltpu.SemaphoreType.DMA,
            pltpu.SemaphoreType.DMA,
            pltpu.SemaphoreType.DMA,
            pltpu.SemaphoreType.DMA,
            pltpu.SemaphoreType.DMA,
        ],
        compiler_params=pltpu.CompilerParams(use_tc_tiling_on_sc=False),
    )
    def edge_kernel(h_hbm, ea_hbm, srcm_hbm, dstm_hbm,
                    out_hbm,
                    srcm_v, dstm_v,
                    hs0, hs1, ea0, ea1, ms0, ms1,
                    buf_v, agg_sh,
                    gsem0, gsem1, esem0, esem1, ssem0, ssem1):
        c = lax.axis_index("c")
        s = lax.axis_index("s")
        wid = c * NS + s
        base = wid * EPT

        zv = jnp.zeros((16,), jnp.float32)

        def zrow(r, carry):
            buf_v[r, pl.ds(0, 16)] = zv
            buf_v[r, pl.ds(16, 16)] = zv
            return carry

        lax.fori_loop(0, RPT, zrow, 0, unroll=8)
        pltpu.sync_copy(buf_v, agg_sh.at[pl.ds(s * RPT, RPT)])

        pltpu.sync_copy(srcm_hbm.at[wid], srcm_v)
        pltpu.sync_copy(dstm_hbm.at[wid], dstm_v)
        plsc.subcore_barrier()

        def ea_slice(e0):
            i = e0 // PRB
            rem = e0 - i * PRB
            cb = rem // PRB4
            row0 = i * PRB4 + (rem - cb * PRB4)
            return row0, cb * H

        def issue(k, eab, hsb, esem, gsem):
            row0, col0 = ea_slice(base + k * CH)
            pltpu.async_copy(
                ea_hbm.at[pl.ds(row0, CH), pl.ds(col0, H)], eab, esem)
            pltpu.async_copy(h_hbm.at[srcm_v.at[k]], hsb, gsem)

        def wait_loads(k, eab, hsb, esem, gsem):
            row0, col0 = ea_slice(base + k * CH)
            pltpu.make_async_copy(
                ea_hbm.at[pl.ds(row0, CH), pl.ds(col0, H)], eab, esem).wait()
            pltpu.make_async_copy(h_hbm.at[srcm_v.at[k]], hsb, gsem).wait()

        def compute(eab, hsb, msb):
            def row(r, rc):
                msb[r, pl.ds(0, 16)] = jnp.maximum(
                    hsb[r, pl.ds(0, 16)] + eab[r, pl.ds(0, 16)], 0.0)
                msb[r, pl.ds(16, 16)] = jnp.maximum(
                    hsb[r, pl.ds(16, 16)] + eab[r, pl.ds(16, 16)], 0.0)
                return rc

            lax.fori_loop(0, CH, row, 0, unroll=8)

        issue(0, ea0, hs0, esem0, gsem0)
        issue(1, ea1, hs1, esem1, gsem1)

        def pair(i, carry):
            k0 = 2 * i
            k1 = 2 * i + 1

            wait_loads(k0, ea0, hs0, esem0, gsem0)

            @pl.when(i > 0)
            def _():
                pltpu.make_async_copy(
                    ms0, agg_sh.at[dstm_v.at[k0 - 2]], ssem0).wait()

            compute(ea0, hs0, ms0)

            @pl.when(k0 + 2 < NCH)
            def _():
                issue(k0 + 2, ea0, hs0, esem0, gsem0)

            pltpu.async_copy(ms0, agg_sh.at[dstm_v.at[k0]], ssem0, add=True)

            wait_loads(k1, ea1, hs1, esem1, gsem1)

            @pl.when(i > 0)
            def _():
                pltpu.make_async_copy(
                    ms1, agg_sh.at[dstm_v.at[k1 - 2]], ssem1).wait()

            compute(ea1, hs1, ms1)

            @pl.when(k1 + 2 < NCH)
            def _():
                issue(k1 + 2, ea1, hs1, esem1, gsem1)

            pltpu.async_copy(ms1, agg_sh.at[dstm_v.at[k1]], ssem1, add=True)
            return carry

        lax.fori_loop(0, NCH // 2, pair, 0)

        pltpu.make_async_copy(ms0, agg_sh.at[dstm_v.at[NCH - 2]], ssem0).wait()
        pltpu.make_async_copy(ms1, agg_sh.at[dstm_v.at[NCH - 1]], ssem1).wait()

        plsc.subcore_barrier()
        pltpu.sync_copy(agg_sh.at[pl.ds(s * RPT, RPT)], buf_v)
        pltpu.sync_copy(buf_v, out_hbm.at[pl.ds(wid * RPT, RPT)])

    return edge_kernel(h, ea, srcm, dstm)


def _proj_node(x, W, b):
    def body(x_ref, w_ref, b_ref, o_ref):
        o_ref[...] = jnp.dot(x_ref[...], w_ref[...],
                             preferred_element_type=jnp.float32) + b_ref[...]

    return pl.pallas_call(
        body, out_shape=jax.ShapeDtypeStruct((N, H), jnp.float32),
    )(x, W, b.reshape(1, H))


def _proj_edge(eaT, W, b):
    RB = PRB
    G = E // RB

    RB4 = RB // 4

    def body(a_ref, w_ref, b_ref, o_ref):
        y = lax.dot_general(a_ref[...], w_ref[...],
                            (((0,), (0,)), ((), ())),
                            preferred_element_type=jnp.float32) + b_ref[...]
        o_ref[...] = jnp.concatenate(
            [y[0:RB4], y[RB4:2 * RB4], y[2 * RB4:3 * RB4], y[3 * RB4:]],
            axis=1)

    return pl.pallas_call(
        body,
        grid=(G,),
        in_specs=[
            pl.BlockSpec((DE, RB), lambda i: (0, i)),
            pl.BlockSpec((DE, H), lambda i: (0, 0)),
            pl.BlockSpec((1, H), lambda i: (0, 0)),
        ],
        out_specs=pl.BlockSpec((RB // 4, 4 * H), lambda i: (i, 0)),
        out_shape=jax.ShapeDtypeStruct((EPAD // 4, 4 * H), jnp.float32),
    )(eaT, W, b.reshape(1, H))


def _dense_layer(h, a0, a1, W1i, b1i, W2i, b2i, gi, bi):
    def body(h_ref, a0_ref, a1_ref, w1, b1, w2, b2, g, bt, o_ref):
        z = h_ref[...] + a0_ref[...] + a1_ref[...]
        t = jnp.maximum(
            jnp.dot(z, w1[...], preferred_element_type=jnp.float32) + b1[...], 0.0)
        z2 = jnp.dot(t, w2[...], preferred_element_type=jnp.float32) + b2[...]
        mu = jnp.mean(z2, axis=0, keepdims=True)
        var = jnp.mean((z2 - mu) ** 2, axis=0, keepdims=True)
        o_ref[...] = jnp.maximum(
            g[...] * (z2 - mu) * lax.rsqrt(var + 1e-5) + bt[...], 0.0)

    return pl.pallas_call(
        body, out_shape=jax.ShapeDtypeStruct((N, H), jnp.float32),
    )(h, a0, a1, W1i, b1i.reshape(1, 2 * H), W2i, b2i.reshape(1, H),
      gi.reshape(1, H), bi.reshape(1, H))


def _dense_final(h, a0, a1, W1i, b1i, W2i, b2i, gi, bi, Wl2, bl2, Wl3, bl3):
    def body(h_ref, a0_ref, a1_ref, w1, b1, w2, b2, g, bt,
             wl2, l2b, wl3, l3b, o_ref):
        z = h_ref[...] + a0_ref[...] + a1_ref[...]
        t = jnp.maximum(
            jnp.dot(z, w1[...], preferred_element_type=jnp.float32) + b1[...], 0.0)
        z2 = jnp.dot(t, w2[...], preferred_element_type=jnp.float32) + b2[...]
        mu = jnp.mean(z2, axis=0, keepdims=True)
        var = jnp.mean((z2 - mu) ** 2, axis=0, keepdims=True)
        hp = jnp.maximum(
            g[...] * (z2 - mu) * lax.rsqrt(var + 1e-5) + bt[...], 0.0)
        gx = jnp.sum(hp, axis=0, keepdims=True)
        p = jnp.maximum(
            jnp.dot(gx, wl2[...], preferred_element_type=jnp.float32) + l2b[...], 0.0)
        o_ref[...] = jnp.dot(p, wl3[...],
                             preferred_element_type=jnp.float32) + l3b[...]

    return pl.pallas_call(
        body, out_shape=jax.ShapeDtypeStruct((1, 1), jnp.float32),
    )(h, a0, a1, W1i, b1i.reshape(1, 2 * H), W2i, b2i.reshape(1, H),
      gi.reshape(1, H), bi.reshape(1, H),
      Wl2, bl2.reshape(1, H // 2), Wl3, bl3.reshape(1, 1))


def kernel(x, edge_attr, edge_index, W_node, b_node, W_edge, b_edge,
           W1, b1, W2, b2, gamma, beta, Wl2, bl2, Wl3, bl3):
    srcp = jnp.concatenate(
        [edge_index[0], jnp.zeros((EPAD - E,), jnp.int32)])
    dstp = jnp.concatenate(
        [edge_index[1], N + jnp.arange(EPAD - E, dtype=jnp.int32) % (NP - N)])
    srcm = srcp.reshape(NW, NCH, CH)
    dstm = dstp.reshape(NW, NCH, CH)
    h = _proj_node(x, W_node, b_node)
    ea = _proj_edge(edge_attr.T, W_edge, b_edge)
    ea = ea.at[E // 4:].set(0.0)
    for i in range(L):
        parts = _edge_pass(h, ea, srcm, dstm)
        a0 = parts[:N]
        a1 = parts[NP:NP + N]
        if i < L - 1:
            h = _dense_layer(h, a0, a1, W1[i], b1[i], W2[i], b2[i],
                             gamma[i], beta[i])
        else:
            res = _dense_final(h, a0, a1, W1[i], b1[i], W2[i], b2[i],
                               gamma[i], beta[i], Wl2, bl2, Wl3, bl3)
    return res.reshape(1)


# speedup vs baseline: 1.0242x; 1.0117x over previous
import functools

import jax
import jax.numpy as jnp
from jax import lax
from jax.experimental import pallas as pl
from jax.experimental.pallas import tpu as pltpu
from jax.experimental.pallas import tpu_sc as plsc

N = 10000
E = 320000
DF = 128
DE = 16
H = 32
L = 3

NC = 2
NS = 16
NW = NC * NS
EPAD = 327680
EPT = EPAD // NW
CH = 128
NCH = EPT // CH
NP = 10240
RPT = NP // NS
PRB = 12800
PRB4 = PRB // 4


def _edge_pass(h, ea, srcm, dstm):
    mesh = plsc.VectorSubcoreMesh(core_axis_name="c", subcore_axis_name="s")

    @functools.partial(
        pl.kernel,
        out_type=jax.ShapeDtypeStruct((NC * NP, H), jnp.float32),
        mesh=mesh,
        scratch_types=[
            pltpu.VMEM((NCH, CH), jnp.int32),
            pltpu.VMEM((NCH, CH), jnp.int32),
            pltpu.VMEM((CH, H), jnp.float32),
            pltpu.VMEM((CH, H), jnp.float32),
            pltpu.VMEM((CH, H), jnp.float32),
            pltpu.VMEM((CH, H), jnp.float32),
            pltpu.VMEM((CH, H), jnp.float32),
            pltpu.VMEM((CH, H), jnp.float32),
            pltpu.VMEM((RPT, H), jnp.float32),
            pltpu.VMEM_SHARED((NP, H), jnp.float32),
            pltpu.SemaphoreType.DMA,
            pltpu.SemaphoreType.DMA,
            pltpu.SemaphoreType.DMA,
            pltpu.SemaphoreType.DMA,
            pltpu.SemaphoreType.DMA,
            pltpu.SemaphoreType.DMA,
        ],
        compiler_params=pltpu.CompilerParams(use_tc_tiling_on_sc=False),
    )
    def edge_kernel(h_hbm, ea_hbm, srcm_hbm, dstm_hbm,
                    out_hbm,
                    srcm_v, dstm_v,
                    hs0, hs1, ea0, ea1, ms0, ms1,
                    buf_v, agg_sh,
                    gsem0, gsem1, esem0, esem1, ssem0, ssem1):
        c = lax.axis_index("c")
        s = lax.axis_index("s")
        wid = c * NS + s
        base = wid * EPT

        zv = jnp.zeros((16,), jnp.float32)

        def zrow(r, carry):
            buf_v[r, pl.ds(0, 16)] = zv
            buf_v[r, pl.ds(16, 16)] = zv
            return carry

        lax.fori_loop(0, RPT, zrow, 0, unroll=8)
        pltpu.sync_copy(buf_v, agg_sh.at[pl.ds(s * RPT, RPT)])

        pltpu.sync_copy(srcm_hbm.at[wid], srcm_v)
        pltpu.sync_copy(dstm_hbm.at[wid], dstm_v)
        plsc.subcore_barrier()

        def ea_slice(e0):
            i = e0 // PRB
            rem = e0 - i * PRB
            cb = rem // PRB4
            row0 = i * PRB4 + (rem - cb * PRB4)
            return row0, cb * H

        def issue(k, eab, hsb, esem, gsem):
            row0, col0 = ea_slice(base + k * CH)
            pltpu.async_copy(
                ea_hbm.at[pl.ds(row0, CH), pl.ds(col0, H)], eab, esem)
            pltpu.async_copy(h_hbm.at[srcm_v.at[k]], hsb, gsem)

        def wait_loads(k, eab, hsb, esem, gsem):
            row0, col0 = ea_slice(base + k * CH)
            pltpu.make_async_copy(
                ea_hbm.at[pl.ds(row0, CH), pl.ds(col0, H)], eab, esem).wait()
            pltpu.make_async_copy(h_hbm.at[srcm_v.at[k]], hsb, gsem).wait()

        def compute(eab, hsb, msb):
            def row(r, rc):
                msb[r, pl.ds(0, 16)] = jnp.maximum(
                    hsb[r, pl.ds(0, 16)] + eab[r, pl.ds(0, 16)], 0.0)
                msb[r, pl.ds(16, 16)] = jnp.maximum(
                    hsb[r, pl.ds(16, 16)] + eab[r, pl.ds(16, 16)], 0.0)
                return rc

            lax.fori_loop(0, CH, row, 0, unroll=8)

        issue(0, ea0, hs0, esem0, gsem0)
        issue(1, ea1, hs1, esem1, gsem1)

        def pair(i, carry):
            k0 = 2 * i
            k1 = 2 * i + 1

            wait_loads(k0, ea0, hs0, esem0, gsem0)

            @pl.when(i > 0)
            def _():
                pltpu.make_async_copy(
                    ms0, agg_sh.at[dstm_v.at[k0 - 2]], ssem0).wait()

            compute(ea0, hs0, ms0)

            @pl.when(k0 + 2 < NCH)
            def _():
                issue(k0 + 2, ea0, hs0, esem0, gsem0)

            pltpu.async_copy(ms0, agg_sh.at[dstm_v.at[k0]], ssem0, add=True)

            wait_loads(k1, ea1, hs1, esem1, gsem1)

            @pl.when(i > 0)
            def _():
                pltpu.make_async_copy(
                    ms1, agg_sh.at[dstm_v.at[k1 - 2]], ssem1).wait()

            compute(ea1, hs1, ms1)

            @pl.when(k1 + 2 < NCH)
            def _():
                issue(k1 + 2, ea1, hs1, esem1, gsem1)

            pltpu.async_copy(ms1, agg_sh.at[dstm_v.at[k1]], ssem1, add=True)
            return carry

        lax.fori_loop(0, NCH // 2, pair, 0)

        pltpu.make_async_copy(ms0, agg_sh.at[dstm_v.at[NCH - 2]], ssem0).wait()
        pltpu.make_async_copy(ms1, agg_sh.at[dstm_v.at[NCH - 1]], ssem1).wait()

        plsc.subcore_barrier()
        pltpu.sync_copy(agg_sh.at[pl.ds(s * RPT, RPT)], buf_v)
        pltpu.sync_copy(buf_v, out_hbm.at[pl.ds(wid * RPT, RPT)])

    return edge_kernel(h, ea, srcm, dstm)


NQ = N // 4


def _sig(n):
    return (n % NQ) * 4 + n // NQ


def _pack4(y):
    return jnp.concatenate(
        [y[0:NQ], y[NQ:2 * NQ], y[2 * NQ:3 * NQ], y[3 * NQ:]], axis=1)


def _proj_node(x, W, b):
    def body(x_ref, w_ref, b_ref, o_ref):
        y = jnp.dot(x_ref[...], w_ref[...],
                    preferred_element_type=jnp.float32) + b_ref[...]
        o_ref[...] = _pack4(y)

    return pl.pallas_call(
        body, out_shape=jax.ShapeDtypeStruct((NQ, 4 * H), jnp.float32),
    )(x, W, b.reshape(1, H))


def _proj_edge(eaT, W, b):
    RB = PRB
    G = E // RB

    RB4 = RB // 4

    def body(a_ref, w_ref, b_ref, o_ref):
        y = lax.dot_general(a_ref[...], w_ref[...],
                            (((0,), (0,)), ((), ())),
                            preferred_element_type=jnp.float32) + b_ref[...]
        o_ref[...] = jnp.concatenate(
            [y[0:RB4], y[RB4:2 * RB4], y[2 * RB4:3 * RB4], y[3 * RB4:]],
            axis=1)

    return pl.pallas_call(
        body,
        grid=(G,),
        in_specs=[
            pl.BlockSpec((DE, RB), lambda i: (0, i)),
            pl.BlockSpec((DE, H), lambda i: (0, 0)),
            pl.BlockSpec((1, H), lambda i: (0, 0)),
        ],
        out_specs=pl.BlockSpec((RB // 4, 4 * H), lambda i: (i, 0)),
        out_shape=jax.ShapeDtypeStruct((EPAD // 4, 4 * H), jnp.float32),
    )(eaT, W, b.reshape(1, H))


def _combine4(s):
    return s[:, 0:H] + s[:, H:2 * H] + s[:, 2 * H:3 * H] + s[:, 3 * H:]


def _tile4(v):
    return jnp.concatenate([v] * 4, axis=1)


def _bn_mlp(h4, p0, p1, w1, b1, w2, b2, g, bt):
    z = h4 + p0 + p1
    t = jnp.maximum(
        jnp.dot(z, w1, preferred_element_type=jnp.float32) + b1, 0.0)
    z2 = jnp.dot(t, w2, preferred_element_type=jnp.float32) + b2
    mu = _tile4(_combine4(jnp.sum(z2, axis=0, keepdims=True)) / N)
    d = z2 - mu
    var = _tile4(_combine4(jnp.sum(d * d, axis=0, keepdims=True)) / N)
    return jnp.maximum(g * d * lax.rsqrt(var + 1e-5) + bt, 0.0)


def _dense_layer(h4, a0, a1, W14, b14, W24, b24, g4, bt4):
    def body(h_ref, a0_ref, a1_ref, w1, b1, w2, b2, g, bt, o_ref):
        o_ref[...] = _bn_mlp(h_ref[...], a0_ref[...], a1_ref[...],
                             w1[...], b1[...], w2[...], b2[...],
                             g[...], bt[...])

    return pl.pallas_call(
        body, out_shape=jax.ShapeDtypeStruct((NQ, 4 * H), jnp.float32),
    )(h4, a0, a1, W14, b14, W24, b24, g4, bt4)


def _dense_final(h4, a0, a1, W14, b14, W24, b24, g4, bt4, Wl2, bl2, Wl3, bl3):
    def body(h_ref, a0_ref, a1_ref, w1, b1, w2, b2, g, bt,
             wl2, l2b, wl3, l3b, o_ref):
        hp = _bn_mlp(h_ref[...], a0_ref[...], a1_ref[...],
                     w1[...], b1[...], w2[...], b2[...], g[...], bt[...])
        gx = _combine4(jnp.sum(hp, axis=0, keepdims=True))
        p = jnp.maximum(
            jnp.dot(gx, wl2[...], preferred_element_type=jnp.float32)
            + l2b[...], 0.0)
        o_ref[...] = jnp.dot(p, wl3[...],
                             preferred_element_type=jnp.float32) + l3b[...]

    return pl.pallas_call(
        body, out_shape=jax.ShapeDtypeStruct((1, 1), jnp.float32),
    )(h4, a0, a1, W14, b14, W24, b24, g4, bt4,
      Wl2, bl2.reshape(1, H // 2), Wl3, bl3.reshape(1, 1))


def kernel(x, edge_attr, edge_index, W_node, b_node, W_edge, b_edge,
           W1, b1, W2, b2, gamma, beta, Wl2, bl2, Wl3, bl3):
    srcp = jnp.concatenate(
        [_sig(edge_index[0]), jnp.zeros((EPAD - E,), jnp.int32)])
    dstp = jnp.concatenate(
        [_sig(edge_index[1]),
         N + jnp.arange(EPAD - E, dtype=jnp.int32) % (NP - N)])
    srcm = srcp.reshape(NW, NCH, CH)
    dstm = dstp.reshape(NW, NCH, CH)
    eye4 = jnp.eye(4, dtype=jnp.float32)
    h4 = _proj_node(x, W_node, b_node)
    ea = _proj_edge(edge_attr.T, W_edge, b_edge)
    ea = ea.at[E // 4:].set(0.0)
    for i in range(L):
        parts = _edge_pass(h4.reshape(N, H), ea, srcm, dstm)
        a0 = parts[:N].reshape(NQ, 4 * H)
        a1 = parts[NP:NP + N].reshape(NQ, 4 * H)
        W14 = jnp.kron(eye4, W1[i])
        b14 = jnp.tile(b1[i], 4).reshape(1, 8 * H)
        W24 = jnp.kron(eye4, W2[i])
        b24 = jnp.tile(b2[i], 4).reshape(1, 4 * H)
        g4 = jnp.tile(gamma[i], 4).reshape(1, 4 * H)
        bt4 = jnp.tile(beta[i], 4).reshape(1, 4 * H)
        if i < L - 1:
            h4 = _dense_layer(h4, a0, a1, W14, b14, W24, b24, g4, bt4)
        else:
            res = _dense_final(h4, a0, a1, W14, b14, W24, b24, g4, bt4,
                               Wl2, bl2, Wl3, bl3)
    return res.reshape(1)
